# Initial kernel scaffold; baseline (speedup 1.0000x reference)
import jax
import jax.numpy as jnp
from jax.experimental import pallas as pl


def kernel(text, label, table):
    raise NotImplementedError("write your pallas kernel here")


# speedup vs baseline: 1.4550x; 1.4550x over previous
import functools

import jax
import jax.numpy as jnp
from jax import lax
from jax.experimental import pallas as pl
from jax.experimental.pallas import tpu as pltpu
from jax.experimental.pallas import tpu_sc as plsc

NUM_CORES = 2
NUM_SUBCORES = 16
NW = NUM_CORES * NUM_SUBCORES

EMBEDDING_DIM = 32
CHUNK = 1024


def _gather_body(idx_hbm, table_hbm, out_hbm, idx_v, rows_v, sem):
    b_total = idx_hbm.shape[0]
    b_per_w = b_total // NW
    n_chunks = b_per_w // CHUNK
    wid = lax.axis_index("s") * NUM_CORES + lax.axis_index("c")
    base = wid * b_per_w

    def body(i, carry):
        off = base + i * CHUNK
        pltpu.sync_copy(idx_hbm.at[pl.ds(off, CHUNK)], idx_v)
        pltpu.async_copy(table_hbm.at[idx_v], rows_v, sem).wait()
        pltpu.sync_copy(rows_v, out_hbm.at[pl.ds(off, CHUNK)])
        return carry

    lax.fori_loop(0, n_chunks, body, 0)


def _embedding_gather(idx_flat, table):
    b_total = idx_flat.shape[0]
    mesh = plsc.VectorSubcoreMesh(core_axis_name="c", subcore_axis_name="s")
    grab = pl.kernel(
        _gather_body,
        out_type=jax.ShapeDtypeStruct((b_total, EMBEDDING_DIM), jnp.float32),
        mesh=mesh,
        scratch_types=[
            pltpu.VMEM((CHUNK,), jnp.int32),
            pltpu.VMEM((CHUNK, EMBEDDING_DIM), jnp.float32),
            pltpu.SemaphoreType.DMA,
        ],
        compiler_params=pltpu.CompilerParams(use_tc_tiling_on_sc=False),
    )
    return grab(idx_flat, table)


def kernel(text, label, table):
    batch, max_len = text.shape
    idx_flat = text.reshape(batch * max_len).astype(jnp.int32)
    rows = _embedding_gather(idx_flat, table)
    return rows.reshape(batch, max_len, EMBEDDING_DIM), label


# speedup vs baseline: 1.4900x; 1.0241x over previous
import jax
import jax.numpy as jnp
from jax import lax
from jax.experimental import pallas as pl
from jax.experimental.pallas import tpu as pltpu
from jax.experimental.pallas import tpu_sc as plsc

NUM_CORES = 2
NUM_SUBCORES = 16
NW = NUM_CORES * NUM_SUBCORES

EMBEDDING_DIM = 32
CHUNK = 1600


def _gather_body(idx_hbm, table_hbm, out_hbm, idx0, idx1, rows0, rows1,
                 gsem0, gsem1, osem0, osem1):
    b_total = idx_hbm.shape[0]
    b_per_w = b_total // NW
    n = b_per_w // CHUNK
    wid = lax.axis_index("s") * NUM_CORES + lax.axis_index("c")
    base = wid * b_per_w

    def start_gather(i, idx_v, rows_v, sem):
        pltpu.sync_copy(idx_hbm.at[pl.ds(base + i * CHUNK, CHUNK)], idx_v)
        pltpu.async_copy(table_hbm.at[idx_v], rows_v, sem)

    def wait_gather(idx_v, rows_v, sem):
        pltpu.make_async_copy(table_hbm.at[idx_v], rows_v, sem).wait()

    def start_out(i, rows_v, sem):
        pltpu.async_copy(rows_v, out_hbm.at[pl.ds(base + i * CHUNK, CHUNK)], sem)

    def wait_out(i, rows_v, sem):
        pltpu.make_async_copy(
            rows_v, out_hbm.at[pl.ds(base + i * CHUNK, CHUNK)], sem).wait()

    start_gather(0, idx0, rows0, gsem0)
    start_gather(1, idx1, rows1, gsem1)
    wait_gather(idx0, rows0, gsem0)
    start_out(0, rows0, osem0)

    def step(j, carry):
        i0 = 2 * j + 1
        wait_out(i0 - 1, rows0, osem0)
        start_gather(i0 + 1, idx0, rows0, gsem0)
        wait_gather(idx1, rows1, gsem1)
        start_out(i0, rows1, osem1)

        i1 = i0 + 1
        wait_out(i1 - 1, rows1, osem1)
        start_gather(i1 + 1, idx1, rows1, gsem1)
        wait_gather(idx0, rows0, gsem0)
        start_out(i1, rows0, osem0)
        return carry

    lax.fori_loop(0, (n - 2) // 2, step, 0)

    wait_gather(idx1, rows1, gsem1)
    start_out(n - 1, rows1, osem1)
    wait_out(n - 2, rows0, osem0)
    wait_out(n - 1, rows1, osem1)


def _embedding_gather(idx_flat, table):
    b_total = idx_flat.shape[0]
    mesh = plsc.VectorSubcoreMesh(core_axis_name="c", subcore_axis_name="s")
    grab = pl.kernel(
        _gather_body,
        out_type=jax.ShapeDtypeStruct((b_total, EMBEDDING_DIM), jnp.float32),
        mesh=mesh,
        scratch_types=[
            pltpu.VMEM((CHUNK,), jnp.int32),
            pltpu.VMEM((CHUNK,), jnp.int32),
            pltpu.VMEM((CHUNK, EMBEDDING_DIM), jnp.float32),
            pltpu.VMEM((CHUNK, EMBEDDING_DIM), jnp.float32),
            pltpu.SemaphoreType.DMA,
            pltpu.SemaphoreType.DMA,
            pltpu.SemaphoreType.DMA,
            pltpu.SemaphoreType.DMA,
        ],
        compiler_params=pltpu.CompilerParams(use_tc_tiling_on_sc=False),
    )
    return grab(idx_flat, table)


def kernel(text, label, table):
    batch, max_len = text.shape
    idx_flat = text.reshape(batch * max_len).astype(jnp.int32)
    rows = _embedding_gather(idx_flat, table)
    return rows.reshape(batch, max_len, EMBEDDING_DIM), label


# speedup vs baseline: 1.4990x; 1.0060x over previous
import jax
import jax.numpy as jnp
from jax import lax
from jax.experimental import pallas as pl
from jax.experimental.pallas import tpu as pltpu
from jax.experimental.pallas import tpu_sc as plsc

NUM_CORES = 2
NUM_SUBCORES = 16
NW = NUM_CORES * NUM_SUBCORES

EMBEDDING_DIM = 32
CHUNK = 800
NBUF = 4


def _gather_body(idx_hbm, table_hbm, out_hbm, idx_v, rows_v, gsem, osem):
    b_total = idx_hbm.shape[0]
    b_per_w = b_total // NW
    n = b_per_w // CHUNK
    wid = lax.axis_index("s") * NUM_CORES + lax.axis_index("c")
    base = wid * b_per_w

    def start_gather(i, b):
        pltpu.sync_copy(idx_hbm.at[pl.ds(base + i * CHUNK, CHUNK)], idx_v.at[b])
        pltpu.async_copy(table_hbm.at[idx_v.at[b]], rows_v.at[b], gsem.at[b])

    def wait_gather(b):
        pltpu.make_async_copy(
            table_hbm.at[idx_v.at[b]], rows_v.at[b], gsem.at[b]).wait()

    def start_out(i, b):
        pltpu.async_copy(
            rows_v.at[b], out_hbm.at[pl.ds(base + i * CHUNK, CHUNK)], osem.at[b])

    def wait_out(i, b):
        pltpu.make_async_copy(
            rows_v.at[b], out_hbm.at[pl.ds(base + i * CHUNK, CHUNK)],
            osem.at[b]).wait()

    for b in range(NBUF - 1):
        start_gather(b, b)
    wait_gather(0)
    start_out(0, 0)
    start_gather(NBUF - 1, NBUF - 1)

    def step(g, carry):
        for b in range(NBUF):
            i = g * NBUF + 1 + b
            p = (1 + b) % NBUF
            q = b % NBUF
            wait_gather(p)
            start_out(i, p)
            wait_out(i - 1, q)
            start_gather(i + NBUF - 1, q)
        return carry

    lax.fori_loop(0, (n - NBUF) // NBUF, step, 0)

    for k in range(NBUF - 1):
        i = n - NBUF + 1 + k
        p = i % NBUF
        wait_gather(p)
        start_out(i, p)
        wait_out(i - 1, (i - 1) % NBUF)
    wait_out(n - 1, (n - 1) % NBUF)


def _embedding_gather(idx_flat, table):
    b_total = idx_flat.shape[0]
    mesh = plsc.VectorSubcoreMesh(core_axis_name="c", subcore_axis_name="s")
    grab = pl.kernel(
        _gather_body,
        out_type=jax.ShapeDtypeStruct((b_total, EMBEDDING_DIM), jnp.float32),
        mesh=mesh,
        scratch_types=[
            pltpu.VMEM((NBUF, CHUNK), jnp.int32),
            pltpu.VMEM((NBUF, CHUNK, EMBEDDING_DIM), jnp.float32),
            pltpu.SemaphoreType.DMA((NBUF,)),
            pltpu.SemaphoreType.DMA((NBUF,)),
        ],
        compiler_params=pltpu.CompilerParams(use_tc_tiling_on_sc=False),
    )
    return grab(idx_flat, table)


def kernel(text, label, table):
    batch, max_len = text.shape
    idx_flat = text.reshape(batch * max_len).astype(jnp.int32)
    rows = _embedding_gather(idx_flat, table)
    return rows.reshape(batch, max_len, EMBEDDING_DIM), label
